# baseline (speedup 1.0000x reference)
import functools

import jax
import jax.numpy as jnp
import numpy as np
from jax.experimental import pallas as pl
from jax.experimental.pallas import tpu as pltpu

_N = 4096
_TOP_K = 10
_L = 40
_R = 256

_DISC = np.where(
    np.arange(_L) < _TOP_K,
    1.0 / np.log2(2.0 + np.arange(_L, dtype=np.float64)),
    0.0,
).astype(np.float32)


def _redress_block_kernel(x_ref, y_ref, o_ref):
    i = pl.program_id(0)
    x = x_ref[...]
    y = y_ref[...]
    lane = jax.lax.broadcasted_iota(jnp.int32, (_R, _N), 1)
    grow = jax.lax.broadcasted_iota(jnp.int32, (_R, _N), 0) + i * _R
    diag = lane == grow
    neg = jnp.float32(-1.0)
    ye = jnp.where(diag, neg, y)
    xe = jnp.where(diag, neg, x)

    ys_cols, yi_cols, xs_cols, xc_cols = [], [], [], []
    for _ in range(_L):
        ym = jnp.max(ye, axis=1, keepdims=True)
        yi = jnp.min(jnp.where(ye == ym, lane, _N), axis=1, keepdims=True)
        oh = lane == yi
        xc = jnp.sum(jnp.where(oh, x, 0.0), axis=1, keepdims=True)
        ye = jnp.where(oh, neg, ye)
        ys_cols.append(ym)
        yi_cols.append(yi.astype(jnp.int32))
        xc_cols.append(xc)
        xm = jnp.max(xe, axis=1, keepdims=True)
        xi = jnp.argmax(xe, axis=1, keepdims=True)
        xe = jnp.where(lane == xi, neg, xe)
        xs_cols.append(xm)

    ys = jnp.concatenate(ys_cols, axis=1)
    yi = jnp.concatenate(yi_cols, axis=1)
    xs = jnp.concatenate(xs_cols, axis=1)
    xc = jnp.concatenate(xc_cols, axis=1)

    pos = jax.lax.broadcasted_iota(jnp.int32, (1, _L), 1).astype(jnp.float32)
    disc = jnp.where(pos < _TOP_K, 1.0 / jnp.log2(2.0 + pos), 0.0)
    idcg = jnp.sum(
        (jnp.exp2(xs[:, :_TOP_K]) - 1.0) * disc[:, :_TOP_K],
        axis=1,
        keepdims=True,
    )
    gain = jnp.exp2(xc) - 1.0

    lam = jnp.zeros((_R, _L), jnp.float32)
    for k in range(_L):
        sx = jnp.sign(xs - xs[:, k : k + 1])
        f1 = -1.0 / (1.0 + jnp.exp(ys - ys[:, k : k + 1]))
        nd = jnp.abs((gain - gain[:, k : k + 1]) * (disc - float(_DISC[k])))
        lam = lam + sx * f1 * nd
    lam = lam / idcg

    out = jnp.zeros((_R, _N), jnp.float32)
    for j in range(_L):
        out = jnp.where(lane == yi[:, j : j + 1], lam[:, j : j + 1], out)
    o_ref[...] = out


@jax.jit
def kernel(x_similarity, y_similarity):
    grid = (_N // _R,)
    return pl.pallas_call(
        _redress_block_kernel,
        grid=grid,
        in_specs=[
            pl.BlockSpec((_R, _N), lambda i: (i, 0)),
            pl.BlockSpec((_R, _N), lambda i: (i, 0)),
        ],
        out_specs=pl.BlockSpec((_R, _N), lambda i: (i, 0)),
        out_shape=jax.ShapeDtypeStruct((_N, _N), jnp.float32),
        compiler_params=pltpu.CompilerParams(
            dimension_semantics=("parallel",)
        ),
    )(x_similarity, y_similarity)


# speedup vs baseline: 1.1653x; 1.1653x over previous
import functools

import jax
import jax.numpy as jnp
import numpy as np
from jax.experimental import pallas as pl
from jax.experimental.pallas import tpu as pltpu

_N = 4096
_TOP_K = 10
_L = 40
_R = 256

_DISC = np.where(
    np.arange(_L) < _TOP_K,
    1.0 / np.log2(2.0 + np.arange(_L, dtype=np.float64)),
    0.0,
).astype(np.float32)


def _redress_block_kernel(x_ref, y_ref, o_ref):
    i = pl.program_id(0)
    x = x_ref[...]
    y = y_ref[...]
    lane = jax.lax.broadcasted_iota(jnp.int32, (_R, _N), 1)
    grow = jax.lax.broadcasted_iota(jnp.int32, (_R, _N), 0) + i * _R
    diag = lane == grow
    neg = jnp.float32(-1.0)
    ye = jnp.where(diag, neg, y)
    xe = jnp.where(diag, neg, x)

    ys_cols, yi_cols, xs_cols, xc_cols = [], [], [], []
    for _ in range(_L):
        ym = jnp.max(ye, axis=1, keepdims=True)
        yi = jnp.min(jnp.where(ye == ym, lane, _N), axis=1, keepdims=True)
        oh = lane == yi
        xc = jnp.sum(jnp.where(oh, x, 0.0), axis=1, keepdims=True)
        ye = jnp.where(oh, neg, ye)
        ys_cols.append(ym)
        yi_cols.append(yi.astype(jnp.int32))
        xc_cols.append(xc)
        xm = jnp.max(xe, axis=1, keepdims=True)
        xi = jnp.min(jnp.where(xe == xm, lane, _N), axis=1, keepdims=True)
        xe = jnp.where(lane == xi, neg, xe)
        xs_cols.append(xm)

    ys = jnp.concatenate(ys_cols, axis=1)
    yi = jnp.concatenate(yi_cols, axis=1)
    xs = jnp.concatenate(xs_cols, axis=1)
    xc = jnp.concatenate(xc_cols, axis=1)

    pos = jax.lax.broadcasted_iota(jnp.int32, (1, _L), 1).astype(jnp.float32)
    disc = jnp.where(pos < _TOP_K, 1.0 / jnp.log2(2.0 + pos), 0.0)
    idcg = jnp.sum(
        (jnp.exp2(xs[:, :_TOP_K]) - 1.0) * disc[:, :_TOP_K],
        axis=1,
        keepdims=True,
    )
    gain = jnp.exp2(xc) - 1.0

    lam = jnp.zeros((_R, _L), jnp.float32)
    for k in range(_L):
        sx = jnp.sign(xs - xs[:, k : k + 1])
        f1 = -1.0 / (1.0 + jnp.exp(ys - ys[:, k : k + 1]))
        nd = jnp.abs((gain - gain[:, k : k + 1]) * (disc - float(_DISC[k])))
        lam = lam + sx * f1 * nd
    lam = lam / idcg

    out = jnp.zeros((_R, _N), jnp.float32)
    for j in range(_L):
        out = jnp.where(lane == yi[:, j : j + 1], lam[:, j : j + 1], out)
    o_ref[...] = out


@jax.jit
def kernel(x_similarity, y_similarity):
    grid = (_N // _R,)
    return pl.pallas_call(
        _redress_block_kernel,
        grid=grid,
        in_specs=[
            pl.BlockSpec((_R, _N), lambda i: (i, 0)),
            pl.BlockSpec((_R, _N), lambda i: (i, 0)),
        ],
        out_specs=pl.BlockSpec((_R, _N), lambda i: (i, 0)),
        out_shape=jax.ShapeDtypeStruct((_N, _N), jnp.float32),
        compiler_params=pltpu.CompilerParams(
            dimension_semantics=("parallel",)
        ),
    )(x_similarity, y_similarity)


# speedup vs baseline: 1.1953x; 1.0257x over previous
import functools

import jax
import jax.numpy as jnp
import numpy as np
from jax.experimental import pallas as pl
from jax.experimental.pallas import tpu as pltpu

_N = 4096
_TOP_K = 10
_L = 40
_R = 256

_DISC = np.where(
    np.arange(_L) < _TOP_K,
    1.0 / np.log2(2.0 + np.arange(_L, dtype=np.float64)),
    0.0,
).astype(np.float32)


def _redress_block_kernel(x_ref, y_ref, o_ref):
    i = pl.program_id(0)
    x = x_ref[...]
    y = y_ref[...]
    lane = jax.lax.broadcasted_iota(jnp.int32, (_R, _N), 1)
    grow = jax.lax.broadcasted_iota(jnp.int32, (_R, _N), 0) + i * _R
    diag = lane == grow
    neg = jnp.float32(-1.0)
    ye = jnp.where(diag, neg, y)
    xe = jnp.where(diag, neg, x)

    ys_cols, yi_cols, xc_cols = [], [], []
    xv_cols, xprev_cols, xcum_cols = [], [], []
    xcum = jnp.zeros((_R, 1), jnp.int32)
    for _ in range(_L):
        ym = jnp.max(ye, axis=1, keepdims=True)
        yi = jnp.min(jnp.where(ye == ym, lane, _N), axis=1, keepdims=True)
        oh = lane == yi
        xc = jnp.sum(jnp.where(oh, x, 0.0), axis=1, keepdims=True)
        ye = jnp.where(oh, neg, ye)
        ys_cols.append(ym)
        yi_cols.append(yi.astype(jnp.int32))
        xc_cols.append(xc)
        xm = jnp.max(xe, axis=1, keepdims=True)
        eqx = xe == xm
        cnt = jnp.sum(eqx.astype(jnp.int32), axis=1, keepdims=True)
        xe = jnp.where(eqx, neg, xe)
        xv_cols.append(xm)
        xprev_cols.append(xcum)
        xcum = xcum + cnt
        xcum_cols.append(xcum)

    ys = jnp.concatenate(ys_cols, axis=1)
    yi = jnp.concatenate(yi_cols, axis=1)
    xc = jnp.concatenate(xc_cols, axis=1)

    pos40 = jax.lax.broadcasted_iota(jnp.int32, (_R, _L), 1)
    xs = jnp.zeros((_R, _L), jnp.float32)
    for t in range(_L):
        m = (xprev_cols[t] <= pos40) & (pos40 < xcum_cols[t])
        xs = jnp.where(m, xv_cols[t], xs)

    pos = jax.lax.broadcasted_iota(jnp.int32, (1, _L), 1).astype(jnp.float32)
    disc = jnp.where(pos < _TOP_K, 1.0 / jnp.log2(2.0 + pos), 0.0)
    idcg = jnp.sum(
        (jnp.exp2(xs[:, :_TOP_K]) - 1.0) * disc[:, :_TOP_K],
        axis=1,
        keepdims=True,
    )
    gain = jnp.exp2(xc) - 1.0

    lam = jnp.zeros((_R, _L), jnp.float32)
    for k in range(_L):
        sx = jnp.sign(xs - xs[:, k : k + 1])
        f1 = -1.0 / (1.0 + jnp.exp(ys - ys[:, k : k + 1]))
        nd = jnp.abs((gain - gain[:, k : k + 1]) * (disc - float(_DISC[k])))
        lam = lam + sx * f1 * nd
    lam = lam / idcg

    out = jnp.zeros((_R, _N), jnp.float32)
    for j in range(_L):
        out = jnp.where(lane == yi[:, j : j + 1], lam[:, j : j + 1], out)
    o_ref[...] = out


@jax.jit
def kernel(x_similarity, y_similarity):
    grid = (_N // _R,)
    return pl.pallas_call(
        _redress_block_kernel,
        grid=grid,
        in_specs=[
            pl.BlockSpec((_R, _N), lambda i: (i, 0)),
            pl.BlockSpec((_R, _N), lambda i: (i, 0)),
        ],
        out_specs=pl.BlockSpec((_R, _N), lambda i: (i, 0)),
        out_shape=jax.ShapeDtypeStruct((_N, _N), jnp.float32),
        compiler_params=pltpu.CompilerParams(
            dimension_semantics=("parallel",)
        ),
    )(x_similarity, y_similarity)
